# baseline (speedup 1.0000x reference)
import jax
import jax.numpy as jnp
from jax.experimental import pallas as pl
from jax.experimental.pallas import tpu as pltpu

_SLAB = 128
_CHUNK = 32
_KB = _SLAB // _CHUNK


def _gmajor(v):
    return jnp.swapaxes(jnp.swapaxes(v, 1, 2), 0, 1)


def _gminor(v):
    return jnp.swapaxes(jnp.swapaxes(v, 0, 1), 1, 2)


def _gconv_kernel(x_ref, w_ref, b_ref, o_ref):
    k = pl.program_id(1)
    for kk in range(_KB):
        @pl.when(k == kk)
        def _(kk=kk):
            lo, hi = kk * _CHUNK, (kk + 1) * _CHUNK
            xt = _gmajor(x_ref[:, :, lo:hi])
            outs = []
            for g in range(_CHUNK):
                og = jax.lax.dot_general(
                    xt[g], w_ref[g],
                    dimension_numbers=(((1,), (1,)), ((), ())),
                    preferred_element_type=jnp.float32,
                )
                outs.append(og + b_ref[g : g + 1, :])
            o_ref[:, :, lo:hi] = _gminor(jnp.stack(outs, axis=0))


def kernel(x, W, b):
    B, Cin, H, Wsp = x.shape
    G = H * Wsp
    Cout = W.shape[1]
    xf = x.reshape(B, Cin, G)
    out = pl.pallas_call(
        _gconv_kernel,
        grid=(G // _SLAB, _KB),
        in_specs=[
            pl.BlockSpec((B, Cin, _SLAB), lambda j, k: (0, 0, j)),
            pl.BlockSpec((_CHUNK, Cout, Cin), lambda j, k: (j * _KB + k, 0, 0)),
            pl.BlockSpec((_CHUNK, Cout), lambda j, k: (j * _KB + k, 0)),
        ],
        out_specs=pl.BlockSpec((B, Cout, _SLAB), lambda j, k: (0, 0, j)),
        out_shape=jax.ShapeDtypeStruct((B, Cout, G), jnp.float32),
        compiler_params=pltpu.CompilerParams(
            dimension_semantics=("parallel", "arbitrary"),
            vmem_limit_bytes=60000 * 1024,
        ),
        name="regroup_conv_fused",
    )(xf, W, b)
    return out.reshape(B, Cout, H, Wsp)


# speedup vs baseline: 1.7779x; 1.7779x over previous
import jax
import jax.numpy as jnp
from jax.experimental import pallas as pl
from jax.experimental.pallas import tpu as pltpu

_GB = 64


def _gconv_kernel(x_ref, w_ref, b_ref, o_ref):
    for g in range(_GB):
        xg = x_ref[g]
        wg = w_ref[g]
        og = jax.lax.dot_general(
            xg, wg,
            dimension_numbers=(((1,), (1,)), ((), ())),
            preferred_element_type=jnp.float32,
        )
        o_ref[g] = og + b_ref[g : g + 1, :]


def kernel(x, W, b):
    B, Cin, H, Wsp = x.shape
    G = H * Wsp
    Cout = W.shape[1]
    xg = jnp.transpose(x, (2, 3, 0, 1)).reshape(G, B, Cin)
    out = pl.pallas_call(
        _gconv_kernel,
        grid=(G // _GB,),
        in_specs=[
            pl.BlockSpec((_GB, B, Cin), lambda j: (j, 0, 0)),
            pl.BlockSpec((_GB, Cout, Cin), lambda j: (j, 0, 0)),
            pl.BlockSpec((_GB, Cout), lambda j: (j, 0)),
        ],
        out_specs=pl.BlockSpec((_GB, B, Cout), lambda j: (j, 0, 0)),
        out_shape=jax.ShapeDtypeStruct((G, B, Cout), jnp.float32),
        compiler_params=pltpu.CompilerParams(
            dimension_semantics=("parallel",),
            vmem_limit_bytes=60000 * 1024,
        ),
        name="regroup_conv_v1",
    )(xg, W, b)
    return jnp.transpose(out, (1, 2, 0)).reshape(B, Cout, H, Wsp)
